# baseline (speedup 1.0000x reference)
import functools

import jax
import jax.numpy as jnp
from jax import lax
from jax.experimental import pallas as pl
from jax.experimental.pallas import tpu as pltpu
from jax.experimental.pallas import tpu_sc as plsc

BATCH = 16384
EMBED = 128
FIELD0 = 100000
NC = 2
NS = 16
NW = NC * NS
BW = BATCH // NW
CH = 64
NCHUNK = BW // CH
NSLOT = 4
LANES = 16

_mesh = plsc.VectorSubcoreMesh(core_axis_name="c", subcore_axis_name="s")


@functools.partial(
    pl.kernel,
    mesh=_mesh,
    out_type=jax.ShapeDtypeStruct((BATCH,), jnp.float32),
    scratch_types=(
        [
            pltpu.VMEM((BW,), jnp.int32),
            pltpu.VMEM((BW,), jnp.int32),
            pltpu.VMEM((BW,), jnp.float32),
            pltpu.VMEM((1024,), jnp.float32),
        ]
        + [pltpu.VMEM((CH, EMBED), jnp.float32) for _ in range(2 * NSLOT)]
        + [pltpu.SemaphoreType.DMA]
    ),
)
def _cmf_fwd(iu_hbm, ii_hbm, table_hbm, out_hbm,
             iu_v, ii_v, ov, shf, *bufs_and_sem):
    ubufs = bufs_and_sem[:NSLOT]
    vbufs = bufs_and_sem[NSLOT:2 * NSLOT]
    sem = bufs_and_sem[2 * NSLOT]

    wid = lax.axis_index("s") * NC + lax.axis_index("c")
    base = wid * BW
    pltpu.sync_copy(iu_hbm.at[pl.ds(base, BW)], iu_v)
    pltpu.sync_copy(ii_hbm.at[pl.ds(base, BW)], ii_v)

    lanes_iota = lax.iota(jnp.int32, LANES)
    bit_masks = [((lanes_iota >> k) & 1) == 1 for k in range(4)]

    def hshift(x, s, center):
        shf[pl.ds(center, LANES)] = x
        return shf[pl.ds(center - s, LANES)]

    def merge(lo, hi, k, m):
        hi2 = hi + hshift(hi, 1 << k, 64 * m + 16)
        lo2 = lo + hshift(lo, -(1 << k), 64 * m + 40)
        return jnp.where(bit_masks[k], hi2, lo2)

    def gather(c):
        slot = c % NSLOT
        cu = pltpu.async_copy(
            table_hbm.at[iu_v.at[pl.ds(c * CH, CH)]], ubufs[slot], sem)
        cv = pltpu.async_copy(
            table_hbm.at[ii_v.at[pl.ds(c * CH, CH)]], vbufs[slot], sem)
        return cu, cv

    pending = [gather(c) for c in range(NSLOT - 1)]
    for c in range(NCHUNK):
        cu, cv = pending.pop(0)
        cu.wait()
        cv.wait()
        if c + NSLOT - 1 < NCHUNK:
            pending.append(gather(c + NSLOT - 1))
        uv = ubufs[c % NSLOT]
        vv = vbufs[c % NSLOT]

        def group(g, _, c=c, uv=uv, vv=vv):
            vecs = []
            for e in range(LANES):
                b = g * LANES + e
                acc = uv[b, pl.ds(0, LANES)] * vv[b, pl.ds(0, LANES)]
                for dj in range(1, EMBED // LANES):
                    acc = acc + uv[b, pl.ds(dj * LANES, LANES)] * vv[b, pl.ds(dj * LANES, LANES)]
                vecs.append(acc)
            m = 0
            for k in range(4):
                nxt = []
                for i in range(len(vecs) // 2):
                    nxt.append(merge(vecs[2 * i], vecs[2 * i + 1], k, m))
                    m += 1
                vecs = nxt
            ov[pl.ds(c * CH + g * LANES, LANES)] = vecs[0]
            return _

        lax.fori_loop(0, CH // LANES, group, 0)

    def sig(j, _):
        z = ov[pl.ds(j * LANES, LANES)]
        ov[pl.ds(j * LANES, LANES)] = 1.0 / (1.0 + jnp.exp(-z))
        return _

    lax.fori_loop(0, BW // LANES, sig, 0)
    pltpu.sync_copy(ov, out_hbm.at[pl.ds(base, BW)])


def kernel(x, table):
    x = x.astype(jnp.int32)
    iu = x[:, 0]
    ii = x[:, 1] + jnp.int32(FIELD0)
    return _cmf_fwd(iu, ii, table)
